# Initial kernel scaffold; baseline (speedup 1.0000x reference)
import jax
import jax.numpy as jnp
from jax.experimental import pallas as pl


def kernel(token_ids, table):
    raise NotImplementedError("write your pallas kernel here")


# speedup vs baseline: 1.5448x; 1.5448x over previous
import functools

import jax
import jax.numpy as jnp
from jax import lax
from jax.experimental import pallas as pl
from jax.experimental.pallas import tpu as pltpu
from jax.experimental.pallas import tpu_sc as plsc

NC = 2
NS = 16
NW = NC * NS
L = 16

B = 4096
S = 200
D = 32
SP = 208
HALF = SP // 2
BPW = B // NW


def _sc_body(tok_flat_hbm, tok3_hbm, table_hbm, out_hbm,
             tokf_v, tok3_v, rows0, rows1, out_v, sem0, sem1):
    cid = lax.axis_index("c")
    sid = lax.axis_index("s")
    wid = sid * NC + cid

    pltpu.sync_copy(tok_flat_hbm.at[wid], tokf_v)
    pltpu.sync_copy(tok3_hbm.at[wid], tok3_v)

    rows = (rows0, rows1)
    sems = (sem0, sem1)

    def start_gather(s, b):
        pltpu.async_copy(table_hbm.at[tok3_v.at[s, 0]],
                         rows[b].at[pl.ds(0, HALF)], sems[b])
        pltpu.async_copy(table_hbm.at[tok3_v.at[s, 1]],
                         rows[b].at[pl.ds(HALF, HALF)], sems[b])

    def wait_gather(b):
        pltpu.make_async_copy(table_hbm.at[pl.ds(0, SP)], rows[b], sems[b]).wait()

    zero = jnp.zeros((L,), jnp.float32)
    izero = jnp.zeros((L,), jnp.int32)

    def compute(s, b):
        rb = rows[b]

        def body4(i, accs):
            a0, a1, a2, a3, a4, a5, a6, a7 = accs
            r = i * 4
            a0 = a0 + rb[r, pl.ds(0, L)]
            a1 = a1 + rb[r, pl.ds(L, L)]
            a2 = a2 + rb[r + 1, pl.ds(0, L)]
            a3 = a3 + rb[r + 1, pl.ds(L, L)]
            a4 = a4 + rb[r + 2, pl.ds(0, L)]
            a5 = a5 + rb[r + 2, pl.ds(L, L)]
            a6 = a6 + rb[r + 3, pl.ds(0, L)]
            a7 = a7 + rb[r + 3, pl.ds(L, L)]
            return (a0, a1, a2, a3, a4, a5, a6, a7)

        a0, a1, a2, a3, a4, a5, a6, a7 = lax.fori_loop(
            0, SP // 4, body4, (zero,) * 8)
        o0 = (a0 + a2) + (a4 + a6)
        o1 = (a1 + a3) + (a5 + a7)

        base = s * SP
        cnt = zero
        for k in range(SP // L):
            t = tokf_v[pl.ds(base + k * L, L)]
            cnt = cnt + jnp.where(t != 0, 1.0, 0.0)

        total = jnp.broadcast_to(jnp.sum(cnt), (L,))
        inv = 1.0 / jnp.maximum(total, 1.0)
        out_v[s, pl.ds(0, L)] = o0 * inv
        out_v[s, pl.ds(L, L)] = o1 * inv

    start_gather(0, 0)
    start_gather(1, 1)

    def group(g, _):
        s0 = g * 2
        wait_gather(0)
        compute(s0, 0)

        @pl.when(s0 + 2 < BPW)
        def _():
            start_gather(s0 + 2, 0)

        wait_gather(1)
        compute(s0 + 1, 1)

        @pl.when(s0 + 3 < BPW)
        def _():
            start_gather(s0 + 3, 1)

        return 0

    lax.fori_loop(0, BPW // 2, group, 0)

    pltpu.sync_copy(out_v, out_hbm.at[pl.ds(wid * BPW, BPW)])


@jax.jit
def _sc_call(tok_flat, tok3, table):
    mesh = plsc.VectorSubcoreMesh(core_axis_name="c", subcore_axis_name="s")
    return pl.kernel(
        _sc_body,
        out_type=jax.ShapeDtypeStruct((B, D), jnp.float32),
        mesh=mesh,
        compiler_params=pltpu.CompilerParams(
            needs_layout_passes=False, use_tc_tiling_on_sc=False),
        scratch_types=[
            pltpu.VMEM((BPW * SP,), jnp.int32),
            pltpu.VMEM((BPW, 2, HALF), jnp.int32),
            pltpu.VMEM((SP, D), jnp.float32),
            pltpu.VMEM((SP, D), jnp.float32),
            pltpu.VMEM((BPW, D), jnp.float32),
            pltpu.SemaphoreType.DMA,
            pltpu.SemaphoreType.DMA,
        ],
    )(tok_flat, tok3, table)


def kernel(token_ids, table):
    tokp = jnp.pad(token_ids, ((0, 0), (0, SP - S)))
    tok_flat = tokp.reshape(NW, BPW * SP)
    tok3 = tokp.reshape(NW, BPW, 2, HALF)
    return _sc_call(tok_flat, tok3, table)


# speedup vs baseline: 2.4244x; 1.5694x over previous
import functools

import jax
import jax.numpy as jnp
from jax import lax
from jax.experimental import pallas as pl
from jax.experimental.pallas import tpu as pltpu
from jax.experimental.pallas import tpu_sc as plsc

NC = 2
NS = 16
NW = NC * NS
L = 16

B = 4096
S = 200
D = 32
BPW = B // NW
G = 4
GS = G * S
NG = BPW // G
STREAMS = ((0, 128), (128, 128), (256, 128), (384, 128),
           (512, 128), (640, 128), (768, 32))


def _sc_body(tok_hbm, table_hbm, out_hbm, tokf_v, rows0, rows1, out_v,
             sem0, sem1):
    cid = lax.axis_index("c")
    sid = lax.axis_index("s")
    wid = sid * NC + cid

    pltpu.sync_copy(tok_hbm.at[wid], tokf_v)

    rows = (rows0, rows1)
    sems = (sem0, sem1)

    def start_gather(g, b):
        tb = g * GS
        for off, ln in STREAMS:
            pltpu.async_copy(table_hbm.at[tokf_v.at[pl.ds(tb + off, ln)]],
                             rows[b].at[pl.ds(off, ln)], sems[b])

    def wait_gather(b):
        pltpu.make_async_copy(table_hbm.at[pl.ds(0, GS)], rows[b], sems[b]).wait()

    zero = jnp.zeros((L,), jnp.float32)
    lane = lax.iota(jnp.int32, 16)
    one = jnp.ones((L,), jnp.float32)

    def compute(g, b):
        rb = rows[b]

        def body2(r, accs):
            accs = list(accs)
            rr = r * 2
            for u in range(G):
                ub = u * S
                for k in range(2):
                    a = u * 4 + k * 2
                    accs[a] = accs[a] + rb[ub + rr + k, pl.ds(0, L)]
                    accs[a + 1] = accs[a + 1] + rb[ub + rr + k, pl.ds(L, L)]
            return tuple(accs)

        accs = lax.fori_loop(0, S // 2, body2, (zero,) * (4 * G))

        for u in range(G):
            s = g * G + u
            o0 = accs[u * 4] + accs[u * 4 + 2]
            o1 = accs[u * 4 + 1] + accs[u * 4 + 3]

            base = s * S
            cnt = zero
            for k in range(S // L):
                t = tokf_v[pl.ds(base + k * L, L)]
                cnt = cnt + jnp.where(t != 0, 1.0, 0.0)
            t = tokf_v[pl.ds(base + S - L, L)]
            cnt = cnt + jnp.where((t != 0) & (lane >= L - S % L), 1.0, 0.0)

            inv = 1.0 / jnp.maximum(jnp.broadcast_to(jnp.sum(cnt), (L,)), one)
            out_v[s, pl.ds(0, L)] = o0 * inv
            out_v[s, pl.ds(L, L)] = o1 * inv

    start_gather(0, 0)
    start_gather(1, 1)

    def pair(i, _):
        g0 = i * 2
        wait_gather(0)
        compute(g0, 0)

        @pl.when(g0 + 2 < NG)
        def _():
            start_gather(g0 + 2, 0)

        wait_gather(1)
        compute(g0 + 1, 1)

        @pl.when(g0 + 3 < NG)
        def _():
            start_gather(g0 + 3, 1)

        return 0

    lax.fori_loop(0, NG // 2, pair, 0)

    pltpu.sync_copy(out_v, out_hbm.at[pl.ds(wid * BPW, BPW)])


@jax.jit
def _sc_call(tok_flat, table):
    mesh = plsc.VectorSubcoreMesh(core_axis_name="c", subcore_axis_name="s")
    return pl.kernel(
        _sc_body,
        out_type=jax.ShapeDtypeStruct((B, D), jnp.float32),
        mesh=mesh,
        compiler_params=pltpu.CompilerParams(
            needs_layout_passes=False, use_tc_tiling_on_sc=False),
        scratch_types=[
            pltpu.VMEM((BPW * S,), jnp.int32),
            pltpu.VMEM((GS, D), jnp.float32),
            pltpu.VMEM((GS, D), jnp.float32),
            pltpu.VMEM((BPW, D), jnp.float32),
            pltpu.SemaphoreType.DMA,
            pltpu.SemaphoreType.DMA,
        ],
    )(tok_flat, table)


def kernel(token_ids, table):
    tok_flat = token_ids.reshape(NW, BPW * S)
    return _sc_call(tok_flat, table)
